# baseline (speedup 1.0000x reference)
import functools

import jax
import jax.numpy as jnp
from jax import lax
from jax.experimental import pallas as pl
from jax.experimental.pallas import tpu as pltpu
from jax.experimental.pallas import tpu_sc as plsc

VOCAB_SIZE = 1000

_NC = 2
_NS = 16
_L = 16
_NW = _NC * _NS


def _lse_body(table_ref, out_ref):
    x = table_ref[...]
    m = jnp.max(x, axis=1, keepdims=True)
    s = jnp.sum(jnp.exp(x - m), axis=1, keepdims=True)
    out_ref[...] = m + jnp.log(s)


def _row_logsumexp(table):
    return pl.pallas_call(
        _lse_body,
        out_shape=jax.ShapeDtypeStruct((table.shape[0], 1), jnp.float32),
    )(table)


def _sc_gather_and_loss(ctx2d, ctx_flat, tgt_flat, table, table_flat, lse):
    bsz, t_len = ctx2d.shape
    n_tok = bsz * t_len
    per_w = n_tok // _NW
    be_per_w = bsz // _NW
    n_grp = per_w // _L
    nbuf = 2

    mesh = plsc.VectorSubcoreMesh(core_axis_name="c", subcore_axis_name="s")

    @functools.partial(
        pl.kernel,
        out_type=[
            jax.ShapeDtypeStruct((bsz, t_len, VOCAB_SIZE), jnp.float32),
            jax.ShapeDtypeStruct((_NW, _L), jnp.float32),
        ],
        mesh=mesh,
        compiler_params=pltpu.CompilerParams(
            needs_layout_passes=False, use_tc_tiling_on_sc=False),
        scratch_types=[
            pltpu.VMEM((be_per_w, t_len), jnp.int32),
            pltpu.VMEM((per_w,), jnp.int32),
            pltpu.VMEM((per_w,), jnp.int32),
            pltpu.VMEM((per_w,), jnp.int32),
            pltpu.VMEM((per_w,), jnp.float32),
            pltpu.VMEM((VOCAB_SIZE,), jnp.float32),
            [pltpu.VMEM((t_len, VOCAB_SIZE), jnp.float32)] * nbuf,
            pltpu.VMEM((_L,), jnp.float32),
            pltpu.SemaphoreType.DMA,
            [pltpu.SemaphoreType.DMA] * nbuf,
            [pltpu.SemaphoreType.DMA] * nbuf,
        ],
    )
    def k(ctx2d_hbm, ctx_hbm, tgt_hbm, table_hbm, tflat_hbm, lse_hbm,
          out_hbm, part_hbm,
          idx2d, idx_all, tgt_all, fidx, picked, lse_v, rows, part_v,
          psem, gsem, ssem):
        wid = lax.axis_index("s") * _NC + lax.axis_index("c")
        base = wid * per_w
        be0 = wid * be_per_w
        pltpu.sync_copy(ctx2d_hbm.at[pl.ds(be0, be_per_w)], idx2d)
        pltpu.sync_copy(ctx_hbm.at[pl.ds(base, per_w)], idx_all)
        pltpu.sync_copy(tgt_hbm.at[pl.ds(base, per_w)], tgt_all)
        pltpu.sync_copy(lse_hbm, lse_v)

        def build(i, _):
            c = idx_all[pl.ds(i * _L, _L)]
            t = tgt_all[pl.ds(i * _L, _L)]
            fidx[pl.ds(i * _L, _L)] = c * VOCAB_SIZE + t
            return 0
        lax.fori_loop(0, n_grp, build, 0)
        pltpu.async_copy(tflat_hbm.at[fidx], picked, psem)

        def gather(g, j):
            pltpu.async_copy(
                table_hbm.at[idx2d.at[g]], rows[j], gsem[j])

        gather(0, 0)
        gather(1, 1)

        def outer_body(outer, _):
            for j in range(nbuf):
                g = outer * nbuf + j
                pltpu.make_async_copy(
                    table_hbm.at[idx2d.at[0]], rows[j], gsem[j]).wait()
                pltpu.async_copy(rows[j], out_hbm.at[be0 + g], ssem[j])

                @pl.when(g + nbuf < be_per_w)
                def _():
                    pltpu.make_async_copy(
                        rows[j], out_hbm.at[0], ssem[j]).wait()
                    gather(g + nbuf, j)
            return 0

        lax.fori_loop(0, be_per_w // nbuf, outer_body, 0)

        for g in (be_per_w - 2, be_per_w - 1):
            j = g % nbuf
            pltpu.make_async_copy(rows[j], out_hbm.at[0], ssem[j]).wait()

        pltpu.make_async_copy(tflat_hbm.at[fidx], picked, psem).wait()

        def acc_body(i, acc):
            c = idx_all[pl.ds(i * _L, _L)]
            lse_g = plsc.load_gather(lse_v, [c])
            p = picked[pl.ds(i * _L, _L)]
            return acc + (lse_g - p)

        acc = lax.fori_loop(0, n_grp, acc_body, jnp.zeros((_L,), jnp.float32))
        part_v[...] = acc
        pltpu.sync_copy(part_v, part_hbm.at[wid])

    return k(ctx2d, ctx_flat, tgt_flat, table, table_flat, lse)


def kernel(batched_context, batched_targets, table):
    b, t = batched_context.shape
    ctx2d = batched_context.astype(jnp.int32)
    ctx_flat = ctx2d.reshape(-1)
    tgt_flat = batched_targets.reshape(-1).astype(jnp.int32)
    table_flat = jnp.concatenate(
        [table.reshape(-1), jnp.zeros((8,), jnp.float32)])

    lse = _row_logsumexp(table)[:, 0]

    logits, part = _sc_gather_and_loss(
        ctx2d, ctx_flat, tgt_flat, table, table_flat, lse)
    loss = jnp.sum(part) / (b * t)
    return (logits, loss)


# speedup vs baseline: 1.2111x; 1.2111x over previous
import functools

import jax
import jax.numpy as jnp
from jax import lax
from jax.experimental import pallas as pl
from jax.experimental.pallas import tpu as pltpu
from jax.experimental.pallas import tpu_sc as plsc

V = 1000
VP = 1024

_NC = 2
_NS = 16
_L = 16
_NW = _NC * _NS


def _lse_body(table_ref, out_ref):
    x = table_ref[...]
    m = jnp.max(x, axis=1, keepdims=True)
    s = jnp.sum(jnp.exp(x - m), axis=1, keepdims=True)
    out_ref[...] = m + jnp.log(s)


def _row_logsumexp(table):
    return pl.pallas_call(
        _lse_body,
        out_shape=jax.ShapeDtypeStruct((table.shape[0], 1), jnp.float32),
    )(table)


def _slice_body(in_ref, out_ref):
    for i in range(out_ref.shape[0]):
        out_ref[i] = in_ref[pl.ds(i * 50, 50), :V]


def _slice_reshape(padded, b, t):
    bb = 8
    return pl.pallas_call(
        _slice_body,
        grid=(b // bb,),
        in_specs=[pl.BlockSpec((bb * t, VP), lambda i: (i, 0))],
        out_specs=pl.BlockSpec((bb, t, V), lambda i: (i, 0, 0)),
        out_shape=jax.ShapeDtypeStruct((b, t, V), jnp.float32),
    )(padded)


def _sc_gather_and_loss(ctx_flat, tgt_flat, table_pad, table_flat, lse):
    n_tok = ctx_flat.shape[0]
    per_w = n_tok // _NW
    n_grp = per_w // _L
    chunk = _L
    n_chunk = per_w // chunk
    nbuf = 4

    mesh = plsc.VectorSubcoreMesh(core_axis_name="c", subcore_axis_name="s")

    @functools.partial(
        pl.kernel,
        out_type=[
            jax.ShapeDtypeStruct((n_tok, VP), jnp.float32),
            jax.ShapeDtypeStruct((_NW * _L,), jnp.float32),
        ],
        mesh=mesh,
        compiler_params=pltpu.CompilerParams(
            needs_layout_passes=False, use_tc_tiling_on_sc=True),
        scratch_types=[
            pltpu.VMEM((per_w,), jnp.int32),
            pltpu.VMEM((per_w,), jnp.int32),
            pltpu.VMEM((per_w,), jnp.int32),
            pltpu.VMEM((per_w,), jnp.float32),
            pltpu.VMEM((V,), jnp.float32),
            [pltpu.VMEM((chunk, VP), jnp.float32)] * nbuf,
            pltpu.VMEM((_L,), jnp.float32),
            pltpu.SemaphoreType.DMA,
            [pltpu.SemaphoreType.DMA] * nbuf,
            [pltpu.SemaphoreType.DMA] * nbuf,
        ],
    )
    def k(ctx_hbm, tgt_hbm, table_hbm, tflat_hbm, lse_hbm,
          out_hbm, part_hbm,
          idx_all, tgt_all, fidx, picked, lse_v, rows, part_v,
          psem, gsem, ssem):
        wid = lax.axis_index("s") * _NC + lax.axis_index("c")
        base = wid * per_w
        pltpu.sync_copy(ctx_hbm.at[pl.ds(base, per_w)], idx_all)
        pltpu.sync_copy(tgt_hbm.at[pl.ds(base, per_w)], tgt_all)
        pltpu.sync_copy(lse_hbm, lse_v)

        def build(i, _):
            c = idx_all[pl.ds(i * _L, _L)]
            t = tgt_all[pl.ds(i * _L, _L)]
            fidx[pl.ds(i * _L, _L)] = c * V + t
            return 0
        lax.fori_loop(0, n_grp, build, 0)
        pltpu.async_copy(tflat_hbm.at[fidx], picked, psem)

        def gather(g, j):
            pltpu.async_copy(
                table_hbm.at[idx_all.at[pl.ds(g * chunk, chunk)]],
                rows[j], gsem[j])

        gather(0, 0)
        gather(1, 1)

        def outer_body(outer, _):
            for j in range(nbuf):
                g = outer * nbuf + j
                jn = (j + 2) % nbuf

                @pl.when(g + 2 < n_chunk)
                def _():
                    @pl.when(g >= 2)
                    def _():
                        pltpu.make_async_copy(
                            rows[jn],
                            out_hbm.at[pl.ds(0, chunk)], ssem[jn]).wait()
                    gather(g + 2, jn)

                pltpu.make_async_copy(
                    table_hbm.at[idx_all.at[pl.ds(0, chunk)]],
                    rows[j], gsem[j]).wait()
                pltpu.async_copy(
                    rows[j], out_hbm.at[pl.ds(base + g * chunk, chunk)],
                    ssem[j])
            return 0

        lax.fori_loop(0, n_chunk // nbuf, outer_body, 0)

        for g in (n_chunk - 2, n_chunk - 1):
            j = g % nbuf
            pltpu.make_async_copy(
                rows[j], out_hbm.at[pl.ds(0, chunk)], ssem[j]).wait()

        pltpu.make_async_copy(tflat_hbm.at[fidx], picked, psem).wait()

        def acc_body(i, acc):
            c = idx_all[pl.ds(i * _L, _L)]
            lse_g = plsc.load_gather(lse_v, [c])
            p = picked[pl.ds(i * _L, _L)]
            return acc + (lse_g - p)

        acc = lax.fori_loop(0, n_grp, acc_body, jnp.zeros((_L,), jnp.float32))
        part_v[...] = acc
        pltpu.sync_copy(part_v, part_hbm.at[pl.ds(wid * _L, _L)])

    return k(ctx_flat, tgt_flat, table_pad, table_flat, lse)


def kernel(batched_context, batched_targets, table):
    b, t = batched_context.shape
    ctx_flat = batched_context.reshape(-1).astype(jnp.int32)
    tgt_flat = batched_targets.reshape(-1).astype(jnp.int32)
    table_pad = jnp.pad(table, ((0, 0), (0, VP - V)))
    table_flat = jnp.concatenate(
        [table.reshape(-1), jnp.zeros((8,), jnp.float32)])

    lse = _row_logsumexp(table)[:, 0]

    padded, part = _sc_gather_and_loss(
        ctx_flat, tgt_flat, table_pad, table_flat, lse)
    logits = _slice_reshape(padded, b, t)
    loss = jnp.sum(part) / (b * t)
    return (logits, loss)


# speedup vs baseline: 1.2653x; 1.0447x over previous
import functools

import jax
import jax.numpy as jnp
from jax import lax
from jax.experimental import pallas as pl
from jax.experimental.pallas import tpu as pltpu
from jax.experimental.pallas import tpu_sc as plsc

V = 1000
B = 1024
T = 50
N_TOK = B * T

_NC = 2
_NS = 16
_L = 16
_NW = _NC * _NS

_C8 = V // 8
_UNITS = T * _C8
_ROWS_PER_T = V * 8
_BG = B // _L


def _lse_body(table_ref, out_ref):
    x = table_ref[...]
    m = jnp.max(x, axis=1, keepdims=True)
    s = jnp.sum(jnp.exp(x - m), axis=1, keepdims=True)
    out_ref[...] = m + jnp.log(s)


def _row_logsumexp(table):
    return pl.pallas_call(
        _lse_body,
        out_shape=jax.ShapeDtypeStruct((table.shape[0], 1), jnp.float32),
    )(table)


def _sc_transposed_gather(ctxT_flat, tgtT_flat, tT128, table_flat, lse):
    per_w = N_TOK // _NW
    n_grp = per_w // _L

    mesh = plsc.VectorSubcoreMesh(core_axis_name="c", subcore_axis_name="s")

    @functools.partial(
        pl.kernel,
        out_type=[
            jax.ShapeDtypeStruct((T * _ROWS_PER_T, 128), jnp.float32),
            jax.ShapeDtypeStruct((_NW * _L,), jnp.float32),
        ],
        mesh=mesh,
        compiler_params=pltpu.CompilerParams(
            needs_layout_passes=False, use_tc_tiling_on_sc=True),
        scratch_types=[
            pltpu.VMEM((N_TOK,), jnp.int32),
            pltpu.VMEM((per_w,), jnp.int32),
            pltpu.VMEM((per_w,), jnp.int32),
            pltpu.VMEM((per_w,), jnp.float32),
            pltpu.VMEM((V,), jnp.float32),
            pltpu.VMEM((64, 128), jnp.float32),
            [pltpu.VMEM((64, 128), jnp.float32)] * 2,
            pltpu.VMEM((_L,), jnp.float32),
            pltpu.SemaphoreType.DMA,
            pltpu.SemaphoreType.DMA,
            [pltpu.SemaphoreType.DMA] * 2,
        ],
    )
    def k(ctx_hbm, tgt_hbm, tt_hbm, tflat_hbm, lse_hbm,
          out_hbm, part_hbm,
          ctx_v, tgt_v, fidx, picked, lse_v, strip, slabs, part_v,
          psem, stsem, ssem):
        wid = lax.axis_index("s") * _NC + lax.axis_index("c")
        base = wid * per_w
        pltpu.sync_copy(ctx_hbm, ctx_v)
        pltpu.sync_copy(tgt_hbm.at[pl.ds(base, per_w)], tgt_v)
        pltpu.sync_copy(lse_hbm, lse_v)

        def build(i, _):
            c = ctx_v[pl.ds(base + i * _L, _L)]
            t = tgt_v[pl.ds(i * _L, _L)]
            fidx[pl.ds(i * _L, _L)] = c * V + t
            return 0
        lax.fori_loop(0, n_grp, build, 0)
        pltpu.async_copy(tflat_hbm.at[fidx], picked, psem)

        u_lo = _UNITS * wid // _NW
        u_hi = _UNITS * (wid + 1) // _NW
        c8_lo = u_lo // T
        c8_hi = (u_hi - 1) // T

        def build_slab(t, c8, j):
            def bg_body(bg, _):
                ctx16 = ctx_v[pl.ds(t * B + bg * _L, _L)]
                hi = jnp.right_shift(ctx16, 7)
                lo = jnp.bitwise_and(ctx16, 127)
                tb = bg // 8
                col = (bg % 8) * _L
                for c_loc in range(8):
                    val = plsc.load_gather(strip, [hi + (c_loc * 8), lo])
                    slabs[j][tb * 8 + c_loc, pl.ds(col, _L)] = val
                return 0
            lax.fori_loop(0, _BG, bg_body, 0)

        def c8_body(c8, _):
            pltpu.async_copy(tt_hbm.at[pl.ds(c8 * 64, 64)], strip, stsem)
            t_lo = jnp.maximum(u_lo - c8 * T, 0)
            t_hi = jnp.minimum(u_hi - c8 * T, T)
            pltpu.make_async_copy(
                tt_hbm.at[pl.ds(0, 64)], strip, stsem).wait()

            def t_body(p, _):
                for j in range(2):
                    i = p * 2 + j
                    t = t_lo + i

                    @pl.when(t < t_hi)
                    def _():
                        @pl.when(i >= 2)
                        def _():
                            pltpu.make_async_copy(
                                slabs[j], out_hbm.at[pl.ds(0, 64)],
                                ssem[j]).wait()
                        build_slab(t, c8, j)
                        pltpu.async_copy(
                            slabs[j],
                            out_hbm.at[pl.ds(t * _ROWS_PER_T + c8 * 64, 64)],
                            ssem[j])
                return 0
            n_t = t_hi - t_lo
            lax.fori_loop(0, (T + 1) // 2, t_body, 0)
            @pl.when(n_t >= 2)
            def _():
                pltpu.make_async_copy(
                    slabs[0], out_hbm.at[pl.ds(0, 64)], ssem[0]).wait()
                pltpu.make_async_copy(
                    slabs[1], out_hbm.at[pl.ds(0, 64)], ssem[1]).wait()
            @pl.when(n_t == 1)
            def _():
                pltpu.make_async_copy(
                    slabs[0], out_hbm.at[pl.ds(0, 64)], ssem[0]).wait()
            return 0

        lax.fori_loop(c8_lo, c8_hi + 1, c8_body, 0)

        pltpu.make_async_copy(tflat_hbm.at[fidx], picked, psem).wait()

        def acc_body(i, acc):
            c = ctx_v[pl.ds(base + i * _L, _L)]
            lse_g = plsc.load_gather(lse_v, [c])
            p = picked[pl.ds(i * _L, _L)]
            return acc + (lse_g - p)

        acc = lax.fori_loop(0, n_grp, acc_body, jnp.zeros((_L,), jnp.float32))
        part_v[...] = acc
        pltpu.sync_copy(part_v, part_hbm.at[pl.ds(wid * _L, _L)])

    return k(ctxT_flat, tgtT_flat, tT128, table_flat, lse)


def kernel(batched_context, batched_targets, table):
    b, t = batched_context.shape
    ctxT_flat = batched_context.astype(jnp.int32).T.reshape(-1)
    tgtT_flat = batched_targets.astype(jnp.int32).T.reshape(-1)
    tT_pad = jnp.pad(table.T, ((0, 0), (0, 24)))
    tT128 = tT_pad.reshape(V * 8, 128)
    table_flat = jnp.concatenate(
        [table.reshape(-1), jnp.zeros((8,), jnp.float32)])

    lse = _row_logsumexp(table)[:, 0]

    out128, part = _sc_transposed_gather(
        ctxT_flat, tgtT_flat, tT128, table_flat, lse)
    out5 = out128.reshape(T, _C8, 8, 8, 128)
    logits = out5.transpose(2, 4, 0, 1, 3).reshape(b, t, V)
    loss = jnp.sum(part) / (b * t)
    return (logits, loss)


# speedup vs baseline: 4.4187x; 3.4922x over previous
import functools

import jax
import jax.numpy as jnp
from jax import lax
from jax.experimental import pallas as pl
from jax.experimental.pallas import tpu as pltpu
from jax.experimental.pallas import tpu_sc as plsc

V = 1000
B = 1024
T = 50
N_TOK = B * T

_NC = 2
_NS = 16
_L = 16
_NW = _NC * _NS

_C8 = V // 8
_UNITS = T * _C8
_ROWS_PER_T = V * 8
_BG = B // _L


def _lse_body(table_ref, out_ref):
    x = table_ref[...]
    m = jnp.max(x, axis=1, keepdims=True)
    s = jnp.sum(jnp.exp(x - m), axis=1, keepdims=True)
    out_ref[...] = m + jnp.log(s)


def _row_logsumexp(table):
    return pl.pallas_call(
        _lse_body,
        out_shape=jax.ShapeDtypeStruct((table.shape[0], 1), jnp.float32),
    )(table)


def _sc_transposed_gather(ctxT_flat, tgtT_flat, tT128, table_flat, lse):
    per_w = N_TOK // _NW
    n_grp = per_w // _L

    mesh = plsc.VectorSubcoreMesh(core_axis_name="c", subcore_axis_name="s")

    @functools.partial(
        pl.kernel,
        out_type=[
            jax.ShapeDtypeStruct((T * _ROWS_PER_T, 128), jnp.float32),
            jax.ShapeDtypeStruct((_NW * _L,), jnp.float32),
        ],
        mesh=mesh,
        compiler_params=pltpu.CompilerParams(
            needs_layout_passes=False, use_tc_tiling_on_sc=True),
        scratch_types=[
            pltpu.VMEM((N_TOK,), jnp.int32),
            pltpu.VMEM((per_w,), jnp.int32),
            pltpu.VMEM((per_w,), jnp.int32),
            pltpu.VMEM((per_w,), jnp.float32),
            pltpu.VMEM((V,), jnp.float32),
            pltpu.VMEM((64, 128), jnp.float32),
            [pltpu.VMEM((64, 128), jnp.float32)] * 2,
            pltpu.VMEM((_L,), jnp.float32),
            pltpu.SemaphoreType.DMA,
            pltpu.SemaphoreType.DMA,
            [pltpu.SemaphoreType.DMA] * 2,
        ],
    )
    def k(ctx_hbm, tgt_hbm, tt_hbm, tflat_hbm, lse_hbm,
          out_hbm, part_hbm,
          ctx_v, tgt_v, fidx, picked, lse_v, strip, slabs, part_v,
          psem, stsem, ssem):
        wid = lax.axis_index("s") * _NC + lax.axis_index("c")
        base = wid * per_w
        pltpu.sync_copy(ctx_hbm, ctx_v)
        pltpu.sync_copy(tgt_hbm.at[pl.ds(base, per_w)], tgt_v)
        pltpu.sync_copy(lse_hbm, lse_v)

        def build(i, _):
            c = ctx_v[pl.ds(base + i * _L, _L)]
            t = tgt_v[pl.ds(i * _L, _L)]
            fidx[pl.ds(i * _L, _L)] = c * V + t
            return 0
        lax.fori_loop(0, n_grp, build, 0)
        pltpu.async_copy(tflat_hbm.at[fidx], picked, psem)

        u_lo = _UNITS * wid // _NW
        u_hi = _UNITS * (wid + 1) // _NW
        c8_lo = u_lo // T
        c8_hi = (u_hi - 1) // T

        def build_slab(t, c8, j):
            @plsc.parallel_loop(0, _BG, unroll=4)
            def bg_body(bg):
                ctx16 = ctx_v[pl.ds(t * B + bg * _L, _L)]
                hi = jnp.right_shift(ctx16, 7)
                lo = jnp.bitwise_and(ctx16, 127)
                tb = bg // 8
                col = (bg % 8) * _L
                for c_loc in range(8):
                    val = plsc.load_gather(strip, [hi + (c_loc * 8), lo])
                    slabs[j][tb * 8 + c_loc, pl.ds(col, _L)] = val

        def c8_body(c8, _):
            pltpu.async_copy(tt_hbm.at[pl.ds(c8 * 64, 64)], strip, stsem)
            t_lo = jnp.maximum(u_lo - c8 * T, 0)
            t_hi = jnp.minimum(u_hi - c8 * T, T)
            pltpu.make_async_copy(
                tt_hbm.at[pl.ds(0, 64)], strip, stsem).wait()

            def t_body(p, _):
                for j in range(2):
                    i = p * 2 + j
                    t = t_lo + i

                    @pl.when(t < t_hi)
                    def _():
                        @pl.when(i >= 2)
                        def _():
                            pltpu.make_async_copy(
                                slabs[j], out_hbm.at[pl.ds(0, 64)],
                                ssem[j]).wait()
                        build_slab(t, c8, j)
                        pltpu.async_copy(
                            slabs[j],
                            out_hbm.at[pl.ds(t * _ROWS_PER_T + c8 * 64, 64)],
                            ssem[j])
                return 0
            n_t = t_hi - t_lo
            lax.fori_loop(0, (T + 1) // 2, t_body, 0)
            @pl.when(n_t >= 2)
            def _():
                pltpu.make_async_copy(
                    slabs[0], out_hbm.at[pl.ds(0, 64)], ssem[0]).wait()
                pltpu.make_async_copy(
                    slabs[1], out_hbm.at[pl.ds(0, 64)], ssem[1]).wait()
            @pl.when(n_t == 1)
            def _():
                pltpu.make_async_copy(
                    slabs[0], out_hbm.at[pl.ds(0, 64)], ssem[0]).wait()
            return 0

        lax.fori_loop(c8_lo, c8_hi + 1, c8_body, 0)

        pltpu.make_async_copy(tflat_hbm.at[fidx], picked, psem).wait()

        def acc_body(i, acc):
            c = ctx_v[pl.ds(base + i * _L, _L)]
            lse_g = plsc.load_gather(lse_v, [c])
            p = picked[pl.ds(i * _L, _L)]
            return acc + (lse_g - p)

        acc = lax.fori_loop(0, n_grp, acc_body, jnp.zeros((_L,), jnp.float32))
        part_v[...] = acc
        pltpu.sync_copy(part_v, part_hbm.at[pl.ds(wid * _L, _L)])

    return k(ctxT_flat, tgtT_flat, tT128, table_flat, lse)


def kernel(batched_context, batched_targets, table):
    b, t = batched_context.shape
    ctxT_flat = batched_context.astype(jnp.int32).T.reshape(-1)
    tgtT_flat = batched_targets.astype(jnp.int32).T.reshape(-1)
    tT_pad = jnp.pad(table.T, ((0, 0), (0, 24)))
    tT128 = tT_pad.reshape(V * 8, 128)
    table_flat = jnp.concatenate(
        [table.reshape(-1), jnp.zeros((8,), jnp.float32)])

    lse = _row_logsumexp(table)[:, 0]

    out128, part = _sc_transposed_gather(
        ctxT_flat, tgtT_flat, tT128, table_flat, lse)
    out5 = out128.reshape(T, _C8, 8, 8, 128)
    logits = out5.transpose(2, 4, 0, 1, 3).reshape(b, t, V)
    loss = jnp.sum(part) / (b * t)
    return (logits, loss)
